# baseline (speedup 1.0000x reference)
import functools

import jax
import jax.numpy as jnp
from jax import lax
from jax.experimental import pallas as pl
from jax.experimental.pallas import tpu as pltpu
from jax.experimental.pallas import tpu_sc as plsc

B = 16384
C = 256
HALF = C // 2
PACK = C // 2

_info = plsc.get_sparse_core_info()
_NC, _NS = _info.num_cores, _info.num_subcores
_NW = _NC * _NS
_B_PER_W = B // _NW
_CHUNK = 128
_N_CHUNKS = _B_PER_W // _CHUNK


def _make_sc_gather(nrows):
    b_per_w = nrows // _NW
    n_chunks = b_per_w // _CHUNK
    mesh = plsc.VectorSubcoreMesh(core_axis_name="c", subcore_axis_name="s")

    @functools.partial(
        pl.kernel,
        mesh=mesh,
        out_type=jax.ShapeDtypeStruct((nrows, PACK), jnp.int32),
        scratch_types=[
            pltpu.VMEM((b_per_w,), jnp.int32),
            pltpu.VMEM((n_chunks, _CHUNK, PACK), jnp.int32),
            pltpu.SemaphoreType.DMA,
            pltpu.SemaphoreType.DMA,
            pltpu.SemaphoreType.DMA,
            pltpu.SemaphoreType.DMA,
            pltpu.SemaphoreType.DMA,
        ],
    )
    def gather_k(idx_hbm, table_hbm, out_hbm, idx_v, rows_v,
                 g0, g1, g2, g3, ssem):
        wid = lax.axis_index("s") * _NC + lax.axis_index("c")
        base = wid * b_per_w
        pltpu.sync_copy(idx_hbm.at[pl.ds(base, b_per_w)], idx_v)
        gsems = (g0, g1, g2, g3)

        g = [
            pltpu.async_copy(
                table_hbm.at[idx_v.at[pl.ds(j * _CHUNK, _CHUNK)]],
                rows_v.at[j],
                gsems[j % 4],
            )
            for j in range(n_chunks)
        ]
        s = []
        for j in range(n_chunks):
            g[j].wait()
            s.append(pltpu.async_copy(
                rows_v.at[j],
                out_hbm.at[pl.ds(base + j * _CHUNK, _CHUNK)],
                ssem,
            ))
        for h in s:
            h.wait()

    return gather_k


HB = B // 2
_sc_gather_half = _make_sc_gather(HB)

_BLK = 4096
_GRPS = _BLK // 128

_INV2PI = 0.15915494309189535
_TPI_HI = 6.28125
_TPI_LO = 0.0019353071795864769
_SIN_C = (1.0, -0.16666666, 8.3333142e-03, -1.9840304e-04,
          2.7532144e-06, -2.4700247e-08, 1.3528492e-10)
_COS_C = (1.0, -0.49999991, 4.1666523e-02, -1.3887963e-03,
          2.4773255e-05, -2.7111616e-07, 1.7362394e-09)


def _poly(c, z):
    acc = jnp.full_like(z, c[-1])
    for k in range(len(c) - 2, -1, -1):
        acc = acc * z + c[k]
    return acc


def _tc_body(timet_ref, invf_ref, emb_ref, out_ref):
    f = invf_ref[...]
    w = emb_ref[...]
    lo = lax.bitcast_convert_type(w << 16, jnp.float32)
    hi = lax.bitcast_convert_type(
        jnp.bitwise_and(w, jnp.int32(-65536)), jnp.float32)
    tcols = jnp.transpose(timet_ref[...])
    for p in range(_GRPS):
        t = tcols[:, p:p + 1]
        x = t * f
        n = (x * _INV2PI + 0.5).astype(jnp.int32).astype(jnp.float32)
        r = (x - n * _TPI_HI) - n * _TPI_LO
        r2 = r * r
        s = r * _poly(_SIN_C, r2)
        c = _poly(_COS_C, r2)
        r0, r1 = p * 128, (p + 1) * 128
        out_ref[r0:r1, :HALF] = s + lo[r0:r1, :]
        out_ref[r0:r1, HALF:] = c + hi[r0:r1, :]


def _tc_body_second(prev_ref, timet_ref, invf_ref, emb_ref, out_ref):
    del prev_ref
    _tc_body(timet_ref, invf_ref, emb_ref, out_ref)


_NBLK_H = HB // _BLK


def _tc_combine_first(timet, inv_freq, emb0):
    return pl.pallas_call(
        _tc_body,
        out_shape=jax.ShapeDtypeStruct((B, C), jnp.float32),
        grid=(_NBLK_H,),
        in_specs=[
            pl.BlockSpec((_GRPS, 128), lambda i: (i, 0)),
            pl.BlockSpec((1, HALF), lambda i: (0, 0)),
            pl.BlockSpec((_BLK, HALF), lambda i: (i, 0)),
        ],
        out_specs=pl.BlockSpec((_BLK, C), lambda i: (i, 0)),
    )(timet, inv_freq, emb0)


def _tc_combine_second(prev, timet, inv_freq, emb1):
    return pl.pallas_call(
        _tc_body_second,
        out_shape=jax.ShapeDtypeStruct((B, C), jnp.float32),
        grid=(_NBLK_H,),
        in_specs=[
            pl.BlockSpec(memory_space=pl.ANY),
            pl.BlockSpec((_GRPS, 128), lambda i: (i + _NBLK_H, 0)),
            pl.BlockSpec((1, HALF), lambda i: (0, 0)),
            pl.BlockSpec((_BLK, HALF), lambda i: (i, 0)),
        ],
        out_specs=pl.BlockSpec((_BLK, C), lambda i: (i + _NBLK_H, 0)),
        input_output_aliases={0: 0},
    )(prev, timet, inv_freq, emb1)


def _round_to_bf16_bits(x):
    u = lax.bitcast_convert_type(x, jnp.int32)
    rounded = u + 0x7FFF + jnp.bitwise_and(lax.shift_right_logical(u, 16), 1)
    return jnp.bitwise_and(lax.shift_right_logical(rounded, 16), 0xFFFF)


def kernel(time, labels, label_emb_table, channels):
    labels_i = labels.astype(jnp.int32)
    lo = _round_to_bf16_bits(label_emb_table[:, :HALF])
    hi = _round_to_bf16_bits(label_emb_table[:, HALF:])
    table_i32 = jnp.bitwise_or(lo, hi << 16)
    emb0 = _sc_gather_half(labels_i[:HB], table_i32)
    emb1 = _sc_gather_half(labels_i[HB:], table_i32)
    timet = time.reshape(B // 128, 128)
    inv_freq = (1.0 / (
        10000.0
        ** (jnp.arange(0, C, 2, dtype=jnp.float32)
            / jnp.asarray(channels).astype(jnp.float32))
    )).reshape(1, HALF)
    out0 = _tc_combine_first(timet, inv_freq, emb0)
    return _tc_combine_second(out0, timet, inv_freq, emb1)


# speedup vs baseline: 1.0860x; 1.0860x over previous
import functools

import jax
import jax.numpy as jnp
from jax import lax
from jax.experimental import pallas as pl
from jax.experimental.pallas import tpu as pltpu
from jax.experimental.pallas import tpu_sc as plsc

B = 16384
C = 256
HALF = C // 2
PACK = C // 2

_info = plsc.get_sparse_core_info()
_NC, _NS = _info.num_cores, _info.num_subcores
_NW = _NC * _NS
_B_PER_W = B // _NW
_CHUNK = 128
_N_CHUNKS = _B_PER_W // _CHUNK


def _make_sc_gather():
    mesh = plsc.VectorSubcoreMesh(core_axis_name="c", subcore_axis_name="s")

    @functools.partial(
        pl.kernel,
        mesh=mesh,
        out_type=jax.ShapeDtypeStruct((B, PACK), jnp.int32),
        scratch_types=[
            pltpu.VMEM((_B_PER_W,), jnp.int32),
            pltpu.VMEM((_N_CHUNKS, _CHUNK, PACK), jnp.int32),
            pltpu.SemaphoreType.DMA,
            pltpu.SemaphoreType.DMA,
            pltpu.SemaphoreType.DMA,
            pltpu.SemaphoreType.DMA,
            pltpu.SemaphoreType.DMA,
        ],
    )
    def gather_k(idx_hbm, table_hbm, out_hbm, idx_v, rows_v,
                 g0, g1, g2, g3, ssem):
        wid = lax.axis_index("s") * _NC + lax.axis_index("c")
        base = wid * _B_PER_W
        pltpu.sync_copy(idx_hbm.at[pl.ds(base, _B_PER_W)], idx_v)
        gsems = (g0, g1, g2, g3)

        g = [
            pltpu.async_copy(
                table_hbm.at[idx_v.at[pl.ds(j * _CHUNK, _CHUNK)]],
                rows_v.at[j],
                gsems[j],
            )
            for j in range(_N_CHUNKS)
        ]
        s = []
        for j in range(_N_CHUNKS):
            g[j].wait()
            s.append(pltpu.async_copy(
                rows_v.at[j],
                out_hbm.at[pl.ds(base + j * _CHUNK, _CHUNK)],
                ssem,
            ))
        for h in s:
            h.wait()

    return gather_k


_sc_gather = _make_sc_gather()

_BLK = 4096
_GRPS = _BLK // 128

_INV2PI = 0.15915494309189535
_TPI_HI = 6.28125
_TPI_LO = 0.0019353071795864769
_SIN_C = (1.0, -0.16666666, 8.3333142e-03, -1.9840304e-04,
          2.7532144e-06, -2.4700247e-08, 1.3528492e-10)
_COS_C = (1.0, -0.49999991, 4.1666523e-02, -1.3887963e-03,
          2.4773255e-05, -2.7111616e-07, 1.7362394e-09)


def _poly(c, z):
    acc = jnp.full_like(z, c[-1])
    for k in range(len(c) - 2, -1, -1):
        acc = acc * z + c[k]
    return acc


def _tc_body(timet_ref, invf_ref, emb_ref, out_ref):
    f = invf_ref[...]
    w = emb_ref[...]
    lo = lax.bitcast_convert_type(w << 16, jnp.float32)
    hi = lax.bitcast_convert_type(
        jnp.bitwise_and(w, jnp.int32(-65536)), jnp.float32)
    tcols = jnp.transpose(timet_ref[...])
    for p in range(_GRPS):
        t = tcols[:, p:p + 1]
        x = t * f
        n = (x * _INV2PI + 0.5).astype(jnp.int32).astype(jnp.float32)
        r = (x - n * _TPI_HI) - n * _TPI_LO
        r2 = r * r
        s = r * _poly(_SIN_C, r2)
        c = _poly(_COS_C, r2)
        r0, r1 = p * 128, (p + 1) * 128
        out_ref[r0:r1, :HALF] = s + lo[r0:r1, :]
        out_ref[r0:r1, HALF:] = c + hi[r0:r1, :]


def _tc_combine(timet, inv_freq, emb):
    return pl.pallas_call(
        _tc_body,
        out_shape=jax.ShapeDtypeStruct((B, C), jnp.float32),
        grid=(B // _BLK,),
        in_specs=[
            pl.BlockSpec((_GRPS, 128), lambda i: (i, 0)),
            pl.BlockSpec((1, HALF), lambda i: (0, 0)),
            pl.BlockSpec((_BLK, HALF), lambda i: (i, 0)),
        ],
        out_specs=pl.BlockSpec((_BLK, C), lambda i: (i, 0)),
    )(timet, inv_freq, emb)


def _round_to_bf16_bits(x):
    u = lax.bitcast_convert_type(x, jnp.int32)
    rounded = u + 0x7FFF + jnp.bitwise_and(lax.shift_right_logical(u, 16), 1)
    return jnp.bitwise_and(lax.shift_right_logical(rounded, 16), 0xFFFF)


def kernel(time, labels, label_emb_table, channels):
    labels_i = labels.astype(jnp.int32)
    lo = _round_to_bf16_bits(label_emb_table[:, :HALF])
    hi = _round_to_bf16_bits(label_emb_table[:, HALF:])
    table_i32 = jnp.bitwise_or(lo, hi << 16)
    emb_i32 = _sc_gather(labels_i, table_i32)
    timet = time.reshape(B // 128, 128)
    inv_freq = (1.0 / (
        10000.0
        ** (jnp.arange(0, C, 2, dtype=jnp.float32)
            / jnp.asarray(channels).astype(jnp.float32))
    )).reshape(1, HALF)
    return _tc_combine(timet, inv_freq, emb_i32)
